# baseline (speedup 1.0000x reference)
import functools

import jax
import jax.numpy as jnp
from jax import lax
from jax.experimental import pallas as pl
from jax.experimental.pallas import tpu as pltpu
from jax.experimental.pallas import tpu_sc as plsc

VOCAB = 1000000
EMBED_DIM = 64
BATCH = 16384
N_FIELDS = 26

TOTAL = BATCH * N_FIELDS
NUM_CORES = 2
NUM_SUBCORES = 16
NUM_WORKERS = NUM_CORES * NUM_SUBCORES
PER_WORKER = TOTAL // NUM_WORKERS
CHUNK = 208
N_CHUNKS = PER_WORKER // CHUNK
SLOTS = 2
GROUPS = CHUNK // 16

_MESH = plsc.VectorSubcoreMesh(core_axis_name="c", subcore_axis_name="s")


@functools.partial(
    pl.kernel,
    mesh=_MESH,
    out_type=jax.ShapeDtypeStruct((TOTAL * EMBED_DIM,), jnp.float32),
    compiler_params=pltpu.CompilerParams(
        use_tc_tiling_on_sc=False, needs_layout_passes=False),
    scratch_types=[
        pltpu.VMEM((PER_WORKER,), jnp.int32),
        pltpu.VMEM((PER_WORKER,), jnp.int32),
        pltpu.VMEM((CHUNK, 2 * EMBED_DIM), jnp.float32),
        pltpu.VMEM((CHUNK, 2 * EMBED_DIM), jnp.float32),
        pltpu.VMEM((CHUNK * EMBED_DIM,), jnp.float32),
        pltpu.VMEM((CHUNK * EMBED_DIM,), jnp.float32),
        pltpu.SemaphoreType.DMA((SLOTS,)),
        pltpu.SemaphoreType.DMA((SLOTS,)),
    ],
)
def _emb_gather(idx_hbm, table_hbm, out_hbm, idx_v, idxp_v, rows0, rows1,
                outb0, outb1, gsems, ssems):
    rows_b = (rows0, rows1)
    outb_b = (outb0, outb1)
    wid = lax.axis_index("s") * NUM_CORES + lax.axis_index("c")
    base = wid * PER_WORKER
    pltpu.sync_copy(idx_hbm.at[pl.ds(base, PER_WORKER)], idx_v)
    iota = lax.iota(jnp.int32, 16)

    def prep(c):
        @plsc.parallel_loop(0, GROUPS, unroll=2)
        def _pbody(g):
            off = c * CHUNK + g * 16
            idxp_v[pl.ds(off, 16)] = idx_v[pl.ds(off, 16)] >> 1

    def gissue(c, par):
        pltpu.async_copy(
            table_hbm.at[idxp_v.at[pl.ds(c * CHUNK, CHUNK)]],
            rows_b[par], gsems.at[par])

    def gwait(par):
        pltpu.make_async_copy(
            table_hbm.at[pl.ds(0, CHUNK)],
            rows_b[par], gsems.at[par]).wait()

    def sissue(c, par):
        pltpu.async_copy(
            outb_b[par],
            out_hbm.at[pl.ds((base + c * CHUNK) * EMBED_DIM,
                             CHUNK * EMBED_DIM)],
            ssems.at[par])

    def swait(par):
        pltpu.make_async_copy(
            outb_b[par],
            out_hbm.at[pl.ds(base * EMBED_DIM, CHUNK * EMBED_DIM)],
            ssems.at[par]).wait()

    def extract(c, par):
        rows2 = rows_b[par]
        ob = outb_b[par]

        @plsc.parallel_loop(0, GROUPS, unroll=2)
        def _gbody(g):
            off = c * CHUNK + g * 16
            hv = idx_v[pl.ds(off, 16)] & 1
            for j in range(16):
                r = g * 16 + j
                hb = jnp.zeros((16,), jnp.int32) + hv[j]
                dst = r << 6
                for k in range(0, EMBED_DIM, 16):
                    lo = rows2[r, pl.ds(k, 16)]
                    hi = rows2[r, pl.ds(EMBED_DIM + k, 16)]
                    ob[pl.ds(dst + k, 16)] = jnp.where(hb == 0, lo, hi)

    prep(0)
    prep(1)
    gissue(0, 0)
    gissue(1, 1)

    def iter_body(i, carry):
        for par in range(SLOTS):
            c = 2 * i + par
            gwait(par)

            @pl.when(c >= SLOTS)
            def _drain():
                swait(par)

            extract(c, par)
            sissue(c, par)

            @pl.when(c + SLOTS < N_CHUNKS)
            def _next():
                prep(c + SLOTS)
                gissue(c + SLOTS, par)
        return carry

    lax.fori_loop(0, N_CHUNKS // 2, iter_body, 0)
    swait(0)
    swait(1)


def kernel(x, weight):
    idx = x.reshape(-1).astype(jnp.int32)
    wt128 = weight.reshape(VOCAB // 2, 2 * EMBED_DIM)
    flat = _emb_gather(idx, wt128)
    return flat.reshape(BATCH, N_FIELDS, EMBED_DIM)


# speedup vs baseline: 1.0862x; 1.0862x over previous
import functools

import jax
import jax.numpy as jnp
from jax import lax
from jax.experimental import pallas as pl
from jax.experimental.pallas import tpu as pltpu
from jax.experimental.pallas import tpu_sc as plsc

VOCAB = 1000000
EMBED_DIM = 64
BATCH = 16384
N_FIELDS = 26

TOTAL = BATCH * N_FIELDS
NUM_CORES = 2
NUM_SUBCORES = 16
NUM_WORKERS = NUM_CORES * NUM_SUBCORES
PER_WORKER = TOTAL // NUM_WORKERS
CHUNK = 416
N_CHUNKS = PER_WORKER // CHUNK
SLOTS = 4

_MESH = plsc.VectorSubcoreMesh(core_axis_name="c", subcore_axis_name="s")


@functools.partial(
    pl.kernel,
    mesh=_MESH,
    out_type=jax.ShapeDtypeStruct((TOTAL, EMBED_DIM), jnp.float32),
    compiler_params=pltpu.CompilerParams(use_tc_tiling_on_sc=False),
    scratch_types=[
        pltpu.VMEM((PER_WORKER,), jnp.int32),
        pltpu.VMEM((SLOTS, CHUNK, EMBED_DIM), jnp.float32),
        pltpu.SemaphoreType.DMA((SLOTS,)),
        pltpu.SemaphoreType.DMA((SLOTS,)),
    ],
)
def _emb_gather(idx_hbm, table_hbm, out_hbm, idx_v, rows_v, gsems, ssems):
    wid = lax.axis_index("s") * NUM_CORES + lax.axis_index("c")
    base = wid * PER_WORKER
    pltpu.sync_copy(idx_hbm.at[pl.ds(base, PER_WORKER)], idx_v)

    def gather(c):
        slot = c % SLOTS
        return pltpu.async_copy(
            table_hbm.at[idx_v.at[pl.ds(c * CHUNK, CHUNK)]],
            rows_v.at[slot], gsems.at[slot])

    def store(c):
        slot = c % SLOTS
        return pltpu.async_copy(
            rows_v.at[slot], out_hbm.at[pl.ds(base + c * CHUNK, CHUNK)],
            ssems.at[slot])

    g = [None] * N_CHUNKS
    s = [None] * N_CHUNKS
    for c in range(SLOTS):
        g[c] = gather(c)
    for c in range(N_CHUNKS):
        g[c].wait()
        s[c] = store(c)
        nxt = c + SLOTS
        if nxt < N_CHUNKS:
            s[c].wait()
            g[nxt] = gather(nxt)
    for c in range(N_CHUNKS - SLOTS, N_CHUNKS):
        s[c].wait()


def kernel(x, weight):
    idx = x.reshape(-1).astype(jnp.int32)
    out = _emb_gather(idx, weight)
    return out.reshape(BATCH, N_FIELDS, EMBED_DIM)
